# Initial kernel scaffold; baseline (speedup 1.0000x reference)
import jax
import jax.numpy as jnp
from jax.experimental import pallas as pl


def kernel(edge_index, node_atts, emb, msg_w_f, msg_b_f, gru_wih_f, gru_whh_f, gru_bih_f, gru_bhh_f, msg_w_b, msg_b_b, gru_wih_b, gru_whh_b, gru_bih_b, gru_bhh_b):
    raise NotImplementedError("write your pallas kernel here")


# speedup vs baseline: 7.6316x; 7.6316x over previous
import functools

import jax
import jax.numpy as jnp
from jax import lax
from jax.experimental import pallas as pl
from jax.experimental.pallas import tpu as pltpu
from jax.experimental.pallas import tpu_sc as plsc

NDIM = 128
HID = 64
L = 3
N = 10000
E = 320000
ATTS = 16
NP_ = 10240

NC = 2
NS = 16
NW = NC * NS
PER_TILE = E // NW
CHUNK = 80
NCHUNKS = PER_TILE // CHUNK
ROWS_PT = NP_ // NS
DW = 16


def _sc_body(compute_deg, *refs):
    if compute_deg:
        (src_hbm, dst_hbm, hf_hbm, hb_hbm, z64_hbm, zdw_hbm, ones_hbm,
         sf_out, sb_out, din_out, dout_out,
         srcb, dstb, rowsf, rowsb, onesv, accf, accb, accdi, accdo,
         semf, semb) = refs
    else:
        (src_hbm, dst_hbm, hf_hbm, hb_hbm, z64_hbm,
         sf_out, sb_out,
         srcb, dstb, rowsf, rowsb, accf, accb,
         semf, semb) = refs

    cid = lax.axis_index("c")
    sid = lax.axis_index("s")
    wid = sid * NC + cid
    rbase = sid * ROWS_PT

    pltpu.sync_copy(z64_hbm, accf.at[pl.ds(rbase, ROWS_PT)])
    pltpu.sync_copy(z64_hbm, accb.at[pl.ds(rbase, ROWS_PT)])
    if compute_deg:
        pltpu.sync_copy(zdw_hbm, accdi.at[pl.ds(rbase, ROWS_PT)])
        pltpu.sync_copy(zdw_hbm, accdo.at[pl.ds(rbase, ROWS_PT)])
        pltpu.sync_copy(ones_hbm, onesv)
    plsc.subcore_barrier()

    ebase = wid * PER_TILE

    @pl.loop(0, NCHUNKS)
    def _chunk(k):
        off = pl.multiple_of(ebase + k * CHUNK, 8)
        pltpu.sync_copy(src_hbm.at[pl.ds(off, CHUNK)], srcb)
        pltpu.sync_copy(dst_hbm.at[pl.ds(off, CHUNK)], dstb)
        gf = pltpu.async_copy(hf_hbm.at[srcb], rowsf, semf)
        gb = pltpu.async_copy(hb_hbm.at[dstb], rowsb, semb)
        gf.wait()
        pltpu.sync_copy(rowsf, accf.at[dstb], add=True)
        gb.wait()
        pltpu.sync_copy(rowsb, accb.at[srcb], add=True)
        if compute_deg:
            pltpu.sync_copy(onesv, accdi.at[dstb], add=True)
            pltpu.sync_copy(onesv, accdo.at[srcb], add=True)

    plsc.subcore_barrier()

    obase = cid * NP_ + rbase
    pltpu.sync_copy(accf.at[pl.ds(rbase, ROWS_PT)],
                    sf_out.at[pl.ds(obase, ROWS_PT)])
    pltpu.sync_copy(accb.at[pl.ds(rbase, ROWS_PT)],
                    sb_out.at[pl.ds(obase, ROWS_PT)])
    if compute_deg:
        pltpu.sync_copy(accdi.at[pl.ds(rbase, ROWS_PT)],
                        din_out.at[pl.ds(obase, ROWS_PT)])
        pltpu.sync_copy(accdo.at[pl.ds(rbase, ROWS_PT)],
                        dout_out.at[pl.ds(obase, ROWS_PT)])


def _make_sc_layer(compute_deg):
    outs = [jax.ShapeDtypeStruct((NC * NP_, HID), jnp.float32),
            jax.ShapeDtypeStruct((NC * NP_, HID), jnp.float32)]
    scratch = [
        pltpu.VMEM((CHUNK,), jnp.int32),
        pltpu.VMEM((CHUNK,), jnp.int32),
        pltpu.VMEM((CHUNK, HID), jnp.float32),
        pltpu.VMEM((CHUNK, HID), jnp.float32),
    ]
    if compute_deg:
        outs += [jax.ShapeDtypeStruct((NC * NP_, DW), jnp.float32),
                 jax.ShapeDtypeStruct((NC * NP_, DW), jnp.float32)]
        scratch.append(pltpu.VMEM((CHUNK, DW), jnp.float32))
    scratch += [
        pltpu.VMEM_SHARED((NP_, HID), jnp.float32),
        pltpu.VMEM_SHARED((NP_, HID), jnp.float32),
    ]
    if compute_deg:
        scratch += [pltpu.VMEM_SHARED((NP_, DW), jnp.float32),
                    pltpu.VMEM_SHARED((NP_, DW), jnp.float32)]
    scratch += [pltpu.SemaphoreType.DMA, pltpu.SemaphoreType.DMA]
    mesh = plsc.VectorSubcoreMesh(core_axis_name="c", subcore_axis_name="s",
                                  num_cores=NC, num_subcores=NS)
    return functools.partial(
        pl.kernel,
        out_type=tuple(outs),
        mesh=mesh,
        scratch_types=scratch,
        compiler_params=pltpu.CompilerParams(use_tc_tiling_on_sc=False),
    )(functools.partial(_sc_body, compute_deg))


@functools.lru_cache(maxsize=None)
def _sc_layers():
    return _make_sc_layer(True), _make_sc_layer(False)


def _rnd(x):
    return x.astype(jnp.bfloat16).astype(jnp.float32)


def _h0_body(atts_ref, emb_ref, out_ref, outr_ref):
    a = atts_ref[...]
    iota = lax.broadcasted_iota(jnp.int32, (a.shape[0], ATTS), 1)
    oh = (a == iota).astype(jnp.float32)
    h0 = jnp.dot(oh, emb_ref[...],
                 precision=lax.Precision.HIGHEST,
                 preferred_element_type=jnp.float32)
    out_ref[...] = h0
    outr_ref[...] = _rnd(h0)


def _gru_update(s0, s1, d0, d1, h, wm, bm, wih, whh, bih, bhh):
    S = s0[...] + s1[...]
    deg = d0[...][:, 0:1] + d1[...][:, 0:1]
    h_ = h[...]
    hr = _rnd(h_)
    wm_ = _rnd(wm[...])
    dn = (((1,), (1,)), ((), ()))
    hp = lax.Precision.HIGHEST
    t = lax.dot_general(hr, wm_[:, HID:], dn, precision=hp,
                        preferred_element_type=jnp.float32) + bm[...]
    aggr = lax.dot_general(S, wm_[:, :HID], dn, precision=hp,
                           preferred_element_type=jnp.float32) + deg * t
    gi = lax.dot_general(_rnd(aggr), _rnd(wih[...]), dn, precision=hp,
                         preferred_element_type=jnp.float32) + bih[...]
    gh = lax.dot_general(hr, _rnd(whh[...]), dn, precision=hp,
                         preferred_element_type=jnp.float32) + bhh[...]
    r = jax.nn.sigmoid(gi[:, :HID] + gh[:, :HID])
    z = jax.nn.sigmoid(gi[:, HID:2 * HID] + gh[:, HID:2 * HID])
    n = jnp.tanh(gi[:, 2 * HID:] + r * gh[:, 2 * HID:])
    return (1.0 - z) * n + z * h_


def _dense_body(sf0, sf1, sb0, sb1, di0, di1, do0, do1, hf, hb,
                wmf, bmf, wihf, whhf, bihf, bhhf,
                wmb, bmb, wihb, whhb, bihb, bhhb,
                hfo, hbo, hfro, hbro):
    hf_new = _gru_update(sf0, sf1, di0, di1, hf,
                         wmf, bmf, wihf, whhf, bihf, bhhf)
    hb_new = _gru_update(sb0, sb1, do0, do1, hb,
                         wmb, bmb, wihb, whhb, bihb, bhhb)
    hfo[...] = hf_new
    hbo[...] = hb_new
    hfro[...] = _rnd(hf_new)
    hbro[...] = _rnd(hb_new)


BN = 1280


def _dense_call(sf, sb, din, dout, hf, hb, wf, wb):
    nblk = NP_ // BN
    part0 = pl.BlockSpec((BN, HID), lambda i: (i, 0))
    part1 = pl.BlockSpec((BN, HID), lambda i: (i + nblk, 0))
    dpart0 = pl.BlockSpec((BN, DW), lambda i: (i, 0))
    dpart1 = pl.BlockSpec((BN, DW), lambda i: (i + nblk, 0))
    hspec = pl.BlockSpec((BN, HID), lambda i: (i, 0))

    def wspec(x):
        return pl.BlockSpec(x.shape, lambda i: tuple(0 for _ in x.shape))

    wmf, bmf, wihf, whhf, bihf, bhhf = wf
    wmb, bmb, wihb, whhb, bihb, bhhb = wb
    weights = (wmf, bmf, wihf, whhf, bihf, bhhf,
               wmb, bmb, wihb, whhb, bihb, bhhb)
    return pl.pallas_call(
        _dense_body,
        grid=(nblk,),
        in_specs=[part0, part1, part0, part1, dpart0, dpart1, dpart0, dpart1,
                  hspec, hspec] + [wspec(w) for w in weights],
        out_specs=[hspec, hspec, hspec, hspec],
        out_shape=[jax.ShapeDtypeStruct((NP_, HID), jnp.float32),
                   jax.ShapeDtypeStruct((NP_, HID), jnp.float32),
                   jax.ShapeDtypeStruct((NP_, HID), jnp.float32),
                   jax.ShapeDtypeStruct((NP_, HID), jnp.float32)],
    )(sf, sf, sb, sb, din, din, dout, dout, hf, hb, *weights)


def kernel(edge_index, node_atts, emb, msg_w_f, msg_b_f, gru_wih_f,
           gru_whh_f, gru_bih_f, gru_bhh_f, msg_w_b, msg_b_b, gru_wih_b,
           gru_whh_b, gru_bih_b, gru_bhh_b):
    src = edge_index[0]
    dst = edge_index[1]

    z64 = jnp.zeros((ROWS_PT, HID), jnp.float32)
    zdw = jnp.zeros((ROWS_PT, DW), jnp.float32)
    ones = jnp.ones((CHUNK, DW), jnp.float32)

    h0, h0r = pl.pallas_call(
        _h0_body,
        grid=(NP_ // BN,),
        in_specs=[pl.BlockSpec((BN, 1), lambda i: (i, 0)),
                  pl.BlockSpec((ATTS, HID), lambda i: (0, 0))],
        out_specs=[pl.BlockSpec((BN, HID), lambda i: (i, 0)),
                   pl.BlockSpec((BN, HID), lambda i: (i, 0))],
        out_shape=[jax.ShapeDtypeStruct((NP_, HID), jnp.float32),
                   jax.ShapeDtypeStruct((NP_, HID), jnp.float32)],
    )(jnp.pad(node_atts, (0, NP_ - N)).reshape(NP_, 1), emb)

    def weights_of(l, wm, bm, wih, whh, bih, bhh):
        return (wm[l], bm[l].reshape(1, NDIM), wih[l], whh[l],
                bih[l].reshape(1, 3 * HID), bhh[l].reshape(1, 3 * HID))

    sc_layer0, sc_layer = _sc_layers()
    hf = hb = h0
    hfr = hbr = h0r
    din = dout = None
    for l in range(L):
        if l == 0:
            sf, sb, din, dout = sc_layer0(src, dst, hfr, hbr, z64, zdw, ones)
        else:
            sf, sb = sc_layer(src, dst, hfr, hbr, z64)
        wf = weights_of(l, msg_w_f, msg_b_f, gru_wih_f, gru_whh_f,
                        gru_bih_f, gru_bhh_f)
        wb = weights_of(l, msg_w_b, msg_b_b, gru_wih_b, gru_whh_b,
                        gru_bih_b, gru_bhh_b)
        hf, hb, hfr, hbr = _dense_call(sf, sb, din, dout, hf, hb, wf, wb)

    return jnp.concatenate([hf[:N], hb[:N]], axis=1)


# speedup vs baseline: 10.3390x; 1.3548x over previous
import functools

import jax
import jax.numpy as jnp
from jax import lax
from jax.experimental import pallas as pl
from jax.experimental.pallas import tpu as pltpu
from jax.experimental.pallas import tpu_sc as plsc

NDIM = 128
HID = 64
L = 3
N = 10000
E = 320000
ATTS = 16
NP_ = 10240

NC = 2
NS = 16
NW = NC * NS
PER_TILE = E // NW
CHUNK = 80
NCHUNKS = PER_TILE // CHUNK
ROWS_PT = NP_ // NS
DW = 16


def _sc_body(compute_deg, *refs):
    if compute_deg:
        (src_hbm, dst_hbm, hf_hbm, hb_hbm, z64_hbm, zdw_hbm, ones_hbm,
         sf_out, sb_out, din_out, dout_out,
         srcb, dstb, rowsf, rowsb, srcb2, dstb2, rowsf2, rowsb2,
         onesv, accf, accb, accdi, accdo,
         semf, semb, semf2, semb2) = refs
    else:
        (src_hbm, dst_hbm, hf_hbm, hb_hbm, z64_hbm,
         sf_out, sb_out,
         srcb, dstb, rowsf, rowsb, srcb2, dstb2, rowsf2, rowsb2,
         accf, accb,
         semf, semb, semf2, semb2) = refs

    cid = lax.axis_index("c")
    sid = lax.axis_index("s")
    wid = sid * NC + cid
    rbase = sid * ROWS_PT

    pltpu.sync_copy(z64_hbm, accf.at[pl.ds(rbase, ROWS_PT)])
    pltpu.sync_copy(z64_hbm, accb.at[pl.ds(rbase, ROWS_PT)])
    if compute_deg:
        pltpu.sync_copy(zdw_hbm, accdi.at[pl.ds(rbase, ROWS_PT)])
        pltpu.sync_copy(zdw_hbm, accdo.at[pl.ds(rbase, ROWS_PT)])
        pltpu.sync_copy(ones_hbm, onesv)
    plsc.subcore_barrier()

    ebase = wid * PER_TILE

    bufA = (srcb, dstb, rowsf, rowsb, semf, semb)
    bufB = (srcb2, dstb2, rowsf2, rowsb2, semf2, semb2)

    def _start(k, sb_, db_, rf_, rb_, smf, smb):
        off = pl.multiple_of(ebase + k * CHUNK, 8)
        pltpu.sync_copy(src_hbm.at[pl.ds(off, CHUNK)], sb_)
        pltpu.sync_copy(dst_hbm.at[pl.ds(off, CHUNK)], db_)
        pltpu.async_copy(hf_hbm.at[sb_], rf_, smf)
        pltpu.async_copy(hb_hbm.at[db_], rb_, smb)

    def _finish(sb_, db_, rf_, rb_, smf, smb):
        pltpu.make_async_copy(hf_hbm.at[sb_], rf_, smf).wait()
        pltpu.sync_copy(rf_, accf.at[db_], add=True)
        pltpu.make_async_copy(hb_hbm.at[db_], rb_, smb).wait()
        pltpu.sync_copy(rb_, accb.at[sb_], add=True)
        if compute_deg:
            pltpu.sync_copy(onesv, accdi.at[db_], add=True)
            pltpu.sync_copy(onesv, accdo.at[sb_], add=True)

    _start(0, *bufA)

    @pl.loop(0, (NCHUNKS - 1) // 2)
    def _pair(j):
        _start(2 * j + 1, *bufB)
        _finish(*bufA)
        _start(2 * j + 2, *bufA)
        _finish(*bufB)

    _finish(*bufA)

    plsc.subcore_barrier()

    obase = cid * NP_ + rbase
    pltpu.sync_copy(accf.at[pl.ds(rbase, ROWS_PT)],
                    sf_out.at[pl.ds(obase, ROWS_PT)])
    pltpu.sync_copy(accb.at[pl.ds(rbase, ROWS_PT)],
                    sb_out.at[pl.ds(obase, ROWS_PT)])
    if compute_deg:
        pltpu.sync_copy(accdi.at[pl.ds(rbase, ROWS_PT)],
                        din_out.at[pl.ds(obase, ROWS_PT)])
        pltpu.sync_copy(accdo.at[pl.ds(rbase, ROWS_PT)],
                        dout_out.at[pl.ds(obase, ROWS_PT)])


def _make_sc_layer(compute_deg):
    outs = [jax.ShapeDtypeStruct((NC * NP_, HID), jnp.float32),
            jax.ShapeDtypeStruct((NC * NP_, HID), jnp.float32)]
    scratch = [
        pltpu.VMEM((CHUNK,), jnp.int32),
        pltpu.VMEM((CHUNK,), jnp.int32),
        pltpu.VMEM((CHUNK, HID), jnp.float32),
        pltpu.VMEM((CHUNK, HID), jnp.float32),
        pltpu.VMEM((CHUNK,), jnp.int32),
        pltpu.VMEM((CHUNK,), jnp.int32),
        pltpu.VMEM((CHUNK, HID), jnp.float32),
        pltpu.VMEM((CHUNK, HID), jnp.float32),
    ]
    if compute_deg:
        outs += [jax.ShapeDtypeStruct((NC * NP_, DW), jnp.float32),
                 jax.ShapeDtypeStruct((NC * NP_, DW), jnp.float32)]
        scratch.append(pltpu.VMEM((CHUNK, DW), jnp.float32))
    scratch += [
        pltpu.VMEM_SHARED((NP_, HID), jnp.float32),
        pltpu.VMEM_SHARED((NP_, HID), jnp.float32),
    ]
    if compute_deg:
        scratch += [pltpu.VMEM_SHARED((NP_, DW), jnp.float32),
                    pltpu.VMEM_SHARED((NP_, DW), jnp.float32)]
    scratch += [pltpu.SemaphoreType.DMA, pltpu.SemaphoreType.DMA,
                pltpu.SemaphoreType.DMA, pltpu.SemaphoreType.DMA]
    mesh = plsc.VectorSubcoreMesh(core_axis_name="c", subcore_axis_name="s",
                                  num_cores=NC, num_subcores=NS)
    return functools.partial(
        pl.kernel,
        out_type=tuple(outs),
        mesh=mesh,
        scratch_types=scratch,
        compiler_params=pltpu.CompilerParams(use_tc_tiling_on_sc=False),
    )(functools.partial(_sc_body, compute_deg))


@functools.lru_cache(maxsize=None)
def _sc_layers():
    return _make_sc_layer(True), _make_sc_layer(False)


def _rnd(x):
    return x.astype(jnp.bfloat16).astype(jnp.float32)


def _h0_body(atts_ref, emb_ref, out_ref, outr_ref):
    a = atts_ref[...]
    iota = lax.broadcasted_iota(jnp.int32, (a.shape[0], ATTS), 1)
    oh = (a == iota).astype(jnp.float32)
    h0 = jnp.dot(oh, emb_ref[...],
                 precision=lax.Precision.HIGHEST,
                 preferred_element_type=jnp.float32)
    out_ref[...] = h0
    outr_ref[...] = _rnd(h0)


def _gru_update(s0, s1, d0, d1, h, wm, bm, wih, whh, bih, bhh):
    S = s0[...] + s1[...]
    deg = d0[...][:, 0:1] + d1[...][:, 0:1]
    h_ = h[...]
    hr = _rnd(h_)
    wm_ = _rnd(wm[...])
    dn = (((1,), (1,)), ((), ()))
    hp = lax.Precision.HIGHEST
    t = lax.dot_general(hr, wm_[:, HID:], dn, precision=hp,
                        preferred_element_type=jnp.float32) + bm[...]
    aggr = lax.dot_general(S, wm_[:, :HID], dn, precision=hp,
                           preferred_element_type=jnp.float32) + deg * t
    gi = lax.dot_general(_rnd(aggr), _rnd(wih[...]), dn, precision=hp,
                         preferred_element_type=jnp.float32) + bih[...]
    gh = lax.dot_general(hr, _rnd(whh[...]), dn, precision=hp,
                         preferred_element_type=jnp.float32) + bhh[...]
    r = jax.nn.sigmoid(gi[:, :HID] + gh[:, :HID])
    z = jax.nn.sigmoid(gi[:, HID:2 * HID] + gh[:, HID:2 * HID])
    n = jnp.tanh(gi[:, 2 * HID:] + r * gh[:, 2 * HID:])
    return (1.0 - z) * n + z * h_


def _dense_body(sf0, sf1, sb0, sb1, di0, di1, do0, do1, hf, hb,
                wmf, bmf, wihf, whhf, bihf, bhhf,
                wmb, bmb, wihb, whhb, bihb, bhhb,
                hfo, hbo, hfro, hbro):
    hf_new = _gru_update(sf0, sf1, di0, di1, hf,
                         wmf, bmf, wihf, whhf, bihf, bhhf)
    hb_new = _gru_update(sb0, sb1, do0, do1, hb,
                         wmb, bmb, wihb, whhb, bihb, bhhb)
    hfo[...] = hf_new
    hbo[...] = hb_new
    hfro[...] = _rnd(hf_new)
    hbro[...] = _rnd(hb_new)


BN = 1280


def _dense_call(sf, sb, din, dout, hf, hb, wf, wb):
    nblk = NP_ // BN
    part0 = pl.BlockSpec((BN, HID), lambda i: (i, 0))
    part1 = pl.BlockSpec((BN, HID), lambda i: (i + nblk, 0))
    dpart0 = pl.BlockSpec((BN, DW), lambda i: (i, 0))
    dpart1 = pl.BlockSpec((BN, DW), lambda i: (i + nblk, 0))
    hspec = pl.BlockSpec((BN, HID), lambda i: (i, 0))

    def wspec(x):
        return pl.BlockSpec(x.shape, lambda i: tuple(0 for _ in x.shape))

    wmf, bmf, wihf, whhf, bihf, bhhf = wf
    wmb, bmb, wihb, whhb, bihb, bhhb = wb
    weights = (wmf, bmf, wihf, whhf, bihf, bhhf,
               wmb, bmb, wihb, whhb, bihb, bhhb)
    return pl.pallas_call(
        _dense_body,
        grid=(nblk,),
        in_specs=[part0, part1, part0, part1, dpart0, dpart1, dpart0, dpart1,
                  hspec, hspec] + [wspec(w) for w in weights],
        out_specs=[hspec, hspec, hspec, hspec],
        out_shape=[jax.ShapeDtypeStruct((NP_, HID), jnp.float32),
                   jax.ShapeDtypeStruct((NP_, HID), jnp.float32),
                   jax.ShapeDtypeStruct((NP_, HID), jnp.float32),
                   jax.ShapeDtypeStruct((NP_, HID), jnp.float32)],
    )(sf, sf, sb, sb, din, din, dout, dout, hf, hb, *weights)


def kernel(edge_index, node_atts, emb, msg_w_f, msg_b_f, gru_wih_f,
           gru_whh_f, gru_bih_f, gru_bhh_f, msg_w_b, msg_b_b, gru_wih_b,
           gru_whh_b, gru_bih_b, gru_bhh_b):
    src = edge_index[0]
    dst = edge_index[1]

    z64 = jnp.zeros((ROWS_PT, HID), jnp.float32)
    zdw = jnp.zeros((ROWS_PT, DW), jnp.float32)
    ones = jnp.ones((CHUNK, DW), jnp.float32)

    h0, h0r = pl.pallas_call(
        _h0_body,
        grid=(NP_ // BN,),
        in_specs=[pl.BlockSpec((BN, 1), lambda i: (i, 0)),
                  pl.BlockSpec((ATTS, HID), lambda i: (0, 0))],
        out_specs=[pl.BlockSpec((BN, HID), lambda i: (i, 0)),
                   pl.BlockSpec((BN, HID), lambda i: (i, 0))],
        out_shape=[jax.ShapeDtypeStruct((NP_, HID), jnp.float32),
                   jax.ShapeDtypeStruct((NP_, HID), jnp.float32)],
    )(jnp.pad(node_atts, (0, NP_ - N)).reshape(NP_, 1), emb)

    def weights_of(l, wm, bm, wih, whh, bih, bhh):
        return (wm[l], bm[l].reshape(1, NDIM), wih[l], whh[l],
                bih[l].reshape(1, 3 * HID), bhh[l].reshape(1, 3 * HID))

    sc_layer0, sc_layer = _sc_layers()
    hf = hb = h0
    hfr = hbr = h0r
    din = dout = None
    for l in range(L):
        if l == 0:
            sf, sb, din, dout = sc_layer0(src, dst, hfr, hbr, z64, zdw, ones)
        else:
            sf, sb = sc_layer(src, dst, hfr, hbr, z64)
        wf = weights_of(l, msg_w_f, msg_b_f, gru_wih_f, gru_whh_f,
                        gru_bih_f, gru_bhh_f)
        wb = weights_of(l, msg_w_b, msg_b_b, gru_wih_b, gru_whh_b,
                        gru_bih_b, gru_bhh_b)
        hf, hb, hfr, hbr = _dense_call(sf, sb, din, dout, hf, hb, wf, wb)

    return jnp.concatenate([hf[:N], hb[:N]], axis=1)
